# baseline (device time: 353736 ns/iter reference)
import jax
import jax.numpy as jnp
from jax import lax
from jax.experimental import pallas as pl
from jax.experimental.pallas import tpu as pltpu

N_DEV = 16
N_HOP = 8
N_SLOT = 4
N_SUB = 8

RING = [0, 1, 5, 9, 13, 14, 10, 6, 2, 3, 7, 11, 15, 12, 8, 4]
POS = [RING.index(i) for i in range(N_DEV)]


def kernel(x, w_mat):
    m_per, k = x.shape
    k2, n_per = w_mat.shape
    assert k == k2
    sub = m_per // N_SUB

    def body(x_ref, w_ref, out_ref,
             cw_comm, ccw_comm,
             cw_send_sems, cw_recv_sems, ccw_send_sems, ccw_recv_sems,
             cw_credit, ccw_credit):
        my = lax.axis_index("i")

        def tlookup(table, idx):
            acc = jnp.int32(table[0])
            for j in range(1, N_DEV):
                acc = jnp.where(idx == j, jnp.int32(table[j]), acc)
            return acc

        my_pos = tlookup(POS, my)

        def ring_at(offset):
            return tlookup(RING, lax.rem(my_pos + offset + 2 * N_DEV, N_DEV))

        left = ring_at(-1)
        right = ring_at(1)
        cw_origin = [ring_at(-h) for h in range(N_HOP)]
        ccw_origin = [ring_at(h) for h in range(N_HOP)]
        origin8 = ring_at(N_HOP)

        barrier_sem = pltpu.get_barrier_semaphore()
        for nbr in (left, right):
            pl.semaphore_signal(
                barrier_sem, inc=1,
                device_id=(nbr,), device_id_type=pl.DeviceIdType.MESH,
            )
        pl.semaphore_wait(barrier_sem, 2)

        def gemm(rows_ref, origin, row_off):
            out_ref[pl.ds(origin * m_per + row_off, rows_ref.shape[0]), :] = (
                jnp.dot(rows_ref[:, :], w_ref[:, :],
                        preferred_element_type=jnp.float32)
            )

        def make_copy(comm, send_sems, recv_sems, src_slot, dst_slot, i, tgt):
            src = (x_ref if src_slot is None else comm.at[src_slot])
            sem_slot = 0 if src_slot is None else src_slot
            return pltpu.make_async_remote_copy(
                src_ref=src.at[pl.ds(i * sub, sub)],
                dst_ref=comm.at[dst_slot, pl.ds(i * sub, sub)],
                send_sem=send_sems.at[sem_slot, i],
                recv_sem=recv_sems.at[dst_slot, i],
                device_id=(tgt,),
                device_id_type=pl.DeviceIdType.MESH,
            )

        prev_cw = None
        prev_ccw = None
        for h in range(N_HOP):
            src_slot = None if h == 0 else h % N_SLOT
            dst_slot = (h + 1) % N_SLOT
            if h >= N_SLOT:
                pl.semaphore_wait(cw_credit, 1)
                pl.semaphore_wait(ccw_credit, 1)

            if h == N_HOP - 1:
                cw_subs = list(range(N_SUB // 2))
                ccw_subs = list(range(N_SUB // 2, N_SUB))
            else:
                cw_subs = ccw_subs = list(range(N_SUB))
            cur_cw, cur_ccw = {}, {}
            for i in cw_subs:
                if h >= 1:
                    prev_cw[i].wait_recv()
                cur_cw[i] = make_copy(cw_comm, cw_send_sems, cw_recv_sems,
                                      src_slot, dst_slot, i, right)
                cur_cw[i].start()
            for i in ccw_subs:
                if h >= 1:
                    prev_ccw[i].wait_recv()
                cur_ccw[i] = make_copy(ccw_comm, ccw_send_sems, ccw_recv_sems,
                                       src_slot, dst_slot, i, left)
                cur_ccw[i].start()

            if h == 0:
                gemm(x_ref, my, 0)
            else:
                if h == N_HOP - 1:
                    for i in range(N_SUB // 2, N_SUB):
                        prev_cw[i].wait_recv()
                    for i in range(N_SUB // 2):
                        prev_ccw[i].wait_recv()
                gemm(cw_comm.at[src_slot], cw_origin[h], 0)
                gemm(ccw_comm.at[src_slot], ccw_origin[h], 0)

            if h >= 1:
                for i in range(N_SUB):
                    prev_cw[i].wait_send()
                    prev_ccw[i].wait_send()
                if 1 <= h - 1 <= N_HOP - N_SLOT:
                    pl.semaphore_signal(
                        cw_credit, inc=1,
                        device_id=(left,), device_id_type=pl.DeviceIdType.MESH,
                    )
                    pl.semaphore_signal(
                        ccw_credit, inc=1,
                        device_id=(right,), device_id_type=pl.DeviceIdType.MESH,
                    )
            prev_cw, prev_ccw = cur_cw, cur_ccw

        last_slot = N_HOP % N_SLOT
        hrows = m_per // 2
        for i in range(N_SUB // 2):
            prev_cw[i].wait_recv()
        gemm(cw_comm.at[last_slot, pl.ds(0, hrows)], origin8, 0)
        for i in range(N_SUB // 2, N_SUB):
            prev_ccw[i].wait_recv()
        gemm(ccw_comm.at[last_slot, pl.ds(hrows, hrows)], origin8, hrows)
        for i in range(N_SUB // 2):
            prev_cw[i].wait_send()
        for i in range(N_SUB // 2, N_SUB):
            prev_ccw[i].wait_send()

    return pl.pallas_call(
        body,
        out_shape=jax.ShapeDtypeStruct((N_DEV * m_per, n_per), jnp.float32),
        in_specs=[
            pl.BlockSpec(memory_space=pltpu.VMEM),
            pl.BlockSpec(memory_space=pltpu.VMEM),
        ],
        out_specs=pl.BlockSpec(memory_space=pltpu.VMEM),
        scratch_shapes=[
            pltpu.VMEM((N_SLOT, m_per, k), jnp.float32),
            pltpu.VMEM((N_SLOT, m_per, k), jnp.float32),
            pltpu.SemaphoreType.DMA((N_SLOT, N_SUB)),
            pltpu.SemaphoreType.DMA((N_SLOT, N_SUB)),
            pltpu.SemaphoreType.DMA((N_SLOT, N_SUB)),
            pltpu.SemaphoreType.DMA((N_SLOT, N_SUB)),
            pltpu.SemaphoreType.REGULAR,
            pltpu.SemaphoreType.REGULAR,
        ],
        compiler_params=pltpu.CompilerParams(collective_id=0),
    )(x, w_mat)


# device time: 353137 ns/iter; 1.0017x vs baseline; 1.0017x over previous
import jax
import jax.numpy as jnp
from jax import lax
from jax.experimental import pallas as pl
from jax.experimental.pallas import tpu as pltpu

N_DEV = 16
N_HOP = 8
N_SLOT = 4
N_SUB = 4

RING = [0, 1, 5, 9, 13, 14, 10, 6, 2, 3, 7, 11, 15, 12, 8, 4]
POS = [RING.index(i) for i in range(N_DEV)]


def kernel(x, w_mat):
    m_per, k = x.shape
    k2, n_per = w_mat.shape
    assert k == k2
    sub = m_per // N_SUB

    def body(x_ref, w_ref, out_ref,
             cw_comm, ccw_comm,
             cw_send_sems, cw_recv_sems, ccw_send_sems, ccw_recv_sems,
             cw_credit, ccw_credit):
        my = lax.axis_index("i")

        def tlookup(table, idx):
            acc = jnp.int32(table[0])
            for j in range(1, N_DEV):
                acc = jnp.where(idx == j, jnp.int32(table[j]), acc)
            return acc

        my_pos = tlookup(POS, my)

        def ring_at(offset):
            return tlookup(RING, lax.rem(my_pos + offset + 2 * N_DEV, N_DEV))

        left = ring_at(-1)
        right = ring_at(1)
        cw_origin = [ring_at(-h) for h in range(N_HOP)]
        ccw_origin = [ring_at(h) for h in range(N_HOP)]
        origin8 = ring_at(N_HOP)

        barrier_sem = pltpu.get_barrier_semaphore()
        for nbr in (left, right):
            pl.semaphore_signal(
                barrier_sem, inc=1,
                device_id=(nbr,), device_id_type=pl.DeviceIdType.MESH,
            )
        pl.semaphore_wait(barrier_sem, 2)

        def gemm(rows_ref, origin, row_off):
            out_ref[pl.ds(origin * m_per + row_off, rows_ref.shape[0]), :] = (
                jnp.dot(rows_ref[:, :], w_ref[:, :],
                        preferred_element_type=jnp.float32)
            )

        def make_copy(comm, send_sems, recv_sems, src_slot, dst_slot, i, tgt):
            src = (x_ref if src_slot is None else comm.at[src_slot])
            sem_slot = 0 if src_slot is None else src_slot
            return pltpu.make_async_remote_copy(
                src_ref=src.at[pl.ds(i * sub, sub)],
                dst_ref=comm.at[dst_slot, pl.ds(i * sub, sub)],
                send_sem=send_sems.at[sem_slot, i],
                recv_sem=recv_sems.at[dst_slot, i],
                device_id=(tgt,),
                device_id_type=pl.DeviceIdType.MESH,
            )

        prev_cw = None
        prev_ccw = None
        for h in range(N_HOP):
            src_slot = None if h == 0 else h % N_SLOT
            dst_slot = (h + 1) % N_SLOT
            if h >= N_SLOT:
                pl.semaphore_wait(cw_credit, 1)
                pl.semaphore_wait(ccw_credit, 1)

            if h == N_HOP - 1:
                cw_subs = list(range(N_SUB // 2))
                ccw_subs = list(range(N_SUB // 2, N_SUB))
            else:
                cw_subs = ccw_subs = list(range(N_SUB))
            cur_cw, cur_ccw = {}, {}
            for i in cw_subs:
                if h >= 1:
                    prev_cw[i].wait_recv()
                cur_cw[i] = make_copy(cw_comm, cw_send_sems, cw_recv_sems,
                                      src_slot, dst_slot, i, right)
                cur_cw[i].start()
            for i in ccw_subs:
                if h >= 1:
                    prev_ccw[i].wait_recv()
                cur_ccw[i] = make_copy(ccw_comm, ccw_send_sems, ccw_recv_sems,
                                       src_slot, dst_slot, i, left)
                cur_ccw[i].start()

            if h == 0:
                gemm(x_ref, my, 0)
            else:
                if h == N_HOP - 1:
                    for i in range(N_SUB // 2, N_SUB):
                        prev_cw[i].wait_recv()
                    for i in range(N_SUB // 2):
                        prev_ccw[i].wait_recv()
                gemm(cw_comm.at[src_slot], cw_origin[h], 0)
                gemm(ccw_comm.at[src_slot], ccw_origin[h], 0)

            if h >= 1:
                for i in range(N_SUB):
                    prev_cw[i].wait_send()
                    prev_ccw[i].wait_send()
                if 1 <= h - 1 <= N_HOP - N_SLOT:
                    pl.semaphore_signal(
                        cw_credit, inc=1,
                        device_id=(left,), device_id_type=pl.DeviceIdType.MESH,
                    )
                    pl.semaphore_signal(
                        ccw_credit, inc=1,
                        device_id=(right,), device_id_type=pl.DeviceIdType.MESH,
                    )
            prev_cw, prev_ccw = cur_cw, cur_ccw

        last_slot = N_HOP % N_SLOT
        hrows = m_per // 2
        for i in range(N_SUB // 2):
            prev_cw[i].wait_recv()
        gemm(cw_comm.at[last_slot, pl.ds(0, hrows)], origin8, 0)
        for i in range(N_SUB // 2, N_SUB):
            prev_ccw[i].wait_recv()
        gemm(ccw_comm.at[last_slot, pl.ds(hrows, hrows)], origin8, hrows)
        for i in range(N_SUB // 2):
            prev_cw[i].wait_send()
        for i in range(N_SUB // 2, N_SUB):
            prev_ccw[i].wait_send()

    return pl.pallas_call(
        body,
        out_shape=jax.ShapeDtypeStruct((N_DEV * m_per, n_per), jnp.float32),
        in_specs=[
            pl.BlockSpec(memory_space=pltpu.VMEM),
            pl.BlockSpec(memory_space=pltpu.VMEM),
        ],
        out_specs=pl.BlockSpec(memory_space=pltpu.VMEM),
        scratch_shapes=[
            pltpu.VMEM((N_SLOT, m_per, k), jnp.float32),
            pltpu.VMEM((N_SLOT, m_per, k), jnp.float32),
            pltpu.SemaphoreType.DMA((N_SLOT, N_SUB)),
            pltpu.SemaphoreType.DMA((N_SLOT, N_SUB)),
            pltpu.SemaphoreType.DMA((N_SLOT, N_SUB)),
            pltpu.SemaphoreType.DMA((N_SLOT, N_SUB)),
            pltpu.SemaphoreType.REGULAR,
            pltpu.SemaphoreType.REGULAR,
        ],
        compiler_params=pltpu.CompilerParams(collective_id=0),
    )(x, w_mat)
